# Initial kernel scaffold; baseline (speedup 1.0000x reference)
import jax
import jax.numpy as jnp
from jax.experimental import pallas as pl


def kernel(edge_index, W1m, b1m, W2m, b2m, W1s, b1s, W2s, b2s):
    raise NotImplementedError("write your pallas kernel here")


# speedup vs baseline: 13.0230x; 13.0230x over previous
import functools
import math

import jax
import jax.numpy as jnp
from jax import lax
from jax.experimental import pallas as pl

NTIPS = 512
HID = 256
BS = 16
NNODES = 2 * NTIPS - 2
DIM = NTIPS - 2
NPAD = 1024
TOL = 1e-5
MAX_ITERS = 10000


def _tree_kernel(idx_ref, pidx_ref, eps_ref, w1t_ref, w1b_ref, b1_ref,
                 w2_ref, b2_ref, samp_ref, logq_ref):
    f32 = jnp.float32
    idxs = idx_ref[0]
    cols = lax.broadcasted_iota(jnp.int32, (NTIPS, NPAD), 1)
    cnt = jnp.zeros((NTIPS, NPAD), f32)
    for k in range(3):
        cnt = cnt + (idxs[:, k:k + 1] == cols).astype(f32)
    M = cnt * (1.0 / 3.0)
    C = M[:, :NTIPS]
    B = M[:, NTIPS:]

    rowmask = (lax.broadcasted_iota(jnp.int32, (NTIPS, 1), 0) < DIM).astype(f32)
    X0 = jnp.full((NTIPS, NTIPS), 1.0 / NTIPS, f32)

    def cond_fn(carry):
        i, _, ln = carry
        return (i < MAX_ITERS) & (ln > TOL)

    def body_fn(carry):
        i, X, _ = carry
        Xn = C + jnp.dot(B, X, preferred_element_type=f32)
        ln = jnp.sum(jnp.abs(Xn - X) * rowmask) * (1.0 / (DIM * NTIPS))
        return i + 1, Xn, ln

    _, X, _ = lax.while_loop(
        cond_fn, body_fn, (jnp.int32(0), X0, jnp.float32(jnp.inf)))

    w1t = w1t_ref[...]
    w1b = w1b_ref[...]
    XT = jnp.dot(X, w1t, preferred_element_type=f32)
    XB = jnp.dot(X, w1b, preferred_element_type=f32)
    childH = jnp.concatenate([w1t, XT], axis=0)
    G = jnp.concatenate([w1b, XB], axis=0)

    pc = pidx_ref[0]
    P = (pc == lax.broadcasted_iota(jnp.int32, (NPAD, NPAD), 1)).astype(f32)
    parentH = jnp.dot(P, G, preferred_element_type=f32)

    H = childH + parentH + b1_ref[...]
    Hact = jnp.where(H > 0, H, jnp.exp(jnp.minimum(H, 0.0)) - 1.0)
    out2 = jnp.dot(Hact, w2_ref[...], preferred_element_type=f32) + b2_ref[...]
    out2t = jnp.transpose(out2)
    mean = out2t[0:1, :]
    colmask = (lax.broadcasted_iota(jnp.int32, (1, NPAD), 1)
               < (NNODES - 1)).astype(f32)
    std = out2t[1:2, :] * colmask
    eps = eps_ref[0]
    samp_ref[0] = eps * jnp.exp(std) + mean - 2.0
    logq0 = jnp.sum((-0.5 * math.log(2 * math.pi) - 0.5 * eps * eps) * colmask)
    logq_ref[0] = jnp.full((1, 128), logq0 - jnp.sum(std), f32)


@jax.jit
def kernel(edge_index, W1m, b1m, W2m, b2m, W1s, b1s, W2s, b2s):
    f32 = jnp.float32
    bs = edge_index.shape[0]
    idx_fix = edge_index[:, NTIPS:, :]
    idx_fix = jnp.pad(idx_fix, ((0, 0), (0, NTIPS - DIM), (0, 0)))
    p_idx = edge_index[:, :NNODES - 1, 0]
    p_idx = jnp.pad(p_idx, ((0, 0), (0, NPAD - (NNODES - 1))))
    p_idx = p_idx[:, :, None]

    W1 = jnp.concatenate([W1m, W1s], axis=1)
    W1_top = W1[:NTIPS]
    W1_bot = W1[NTIPS:]
    b1 = jnp.concatenate([b1m, b1s])[None, :]
    W2 = jnp.zeros((2 * HID, 2), f32)
    W2 = W2.at[:HID, 0].set(W2m[:, 0]).at[HID:, 1].set(W2s[:, 0])
    b2 = jnp.stack([b2m[0], b2s[0]])[None, :]

    eps = jax.random.normal(jax.random.key(42), (bs, NNODES - 1), dtype=f32)
    eps_p = jnp.pad(eps, ((0, 0), (0, NPAD - (NNODES - 1))))[:, None, :]

    grid = (bs,)
    samp_out, logq_out = pl.pallas_call(
        _tree_kernel,
        grid=grid,
        in_specs=[
            pl.BlockSpec((1, NTIPS, 3), lambda b: (b, 0, 0)),
            pl.BlockSpec((1, NPAD, 1), lambda b: (b, 0, 0)),
            pl.BlockSpec((1, 1, NPAD), lambda b: (b, 0, 0)),
            pl.BlockSpec((NTIPS, NTIPS), lambda b: (0, 0)),
            pl.BlockSpec((NTIPS, NTIPS), lambda b: (0, 0)),
            pl.BlockSpec((1, 2 * HID), lambda b: (0, 0)),
            pl.BlockSpec((2 * HID, 2), lambda b: (0, 0)),
            pl.BlockSpec((1, 2), lambda b: (0, 0)),
        ],
        out_specs=[
            pl.BlockSpec((1, 1, NPAD), lambda b: (b, 0, 0)),
            pl.BlockSpec((1, 1, 128), lambda b: (b, 0, 0)),
        ],
        out_shape=[
            jax.ShapeDtypeStruct((bs, 1, NPAD), f32),
            jax.ShapeDtypeStruct((bs, 1, 128), f32),
        ],
    )(idx_fix, p_idx, eps_p, W1_top, W1_bot, b1, W2, b2)

    samp_log_branch = samp_out[:, 0, :NNODES - 1]
    logq_branch = logq_out[:, 0, 0]
    return samp_log_branch, logq_branch


# speedup vs baseline: 14.7403x; 1.1319x over previous
import functools
import math

import jax
import jax.numpy as jnp
from jax import lax
from jax.experimental import pallas as pl

NTIPS = 512
HID = 256
BS = 16
NNODES = 2 * NTIPS - 2
DIM = NTIPS - 2
NPAD = 1024
TOL = 1e-5
MAX_ITERS = 10000


def _tree_kernel(idx_ref, pidx_ref, eps_ref, w1t_ref, w1b_ref, b1_ref,
                 w2_ref, b2_ref, samp_ref, logq_ref):
    f32 = jnp.float32
    idxs = idx_ref[0]
    cols = lax.broadcasted_iota(jnp.int32, (NTIPS, NPAD), 1)
    cnt = jnp.zeros((NTIPS, NPAD), f32)
    for k in range(3):
        cnt = cnt + (idxs[:, k:k + 1] == cols).astype(f32)
    M = cnt * (1.0 / 3.0)
    C = M[:, :NTIPS]
    B = M[:, NTIPS:]

    X0 = jnp.full((NTIPS, NTIPS), 1.0 / NTIPS, f32)

    def cond_fn(carry):
        i, _, ln = carry
        return (i < MAX_ITERS) & (ln > TOL)

    def body_fn(carry):
        i, X, _ = carry
        X1 = C + jnp.dot(B, X, preferred_element_type=f32)
        X2 = C + jnp.dot(B, X1, preferred_element_type=f32)
        ln = jnp.sum(jnp.abs(X2 - X1)) * (1.0 / (DIM * NTIPS))
        return i + 2, X2, ln

    _, X, _ = lax.while_loop(
        cond_fn, body_fn, (jnp.int32(0), X0, jnp.float32(jnp.inf)))

    bf = jnp.bfloat16
    w1t = w1t_ref[...]
    w1b = w1b_ref[...]
    Xb = X.astype(bf)
    XT = jnp.dot(Xb, w1t.astype(bf), preferred_element_type=f32)
    XB = jnp.dot(Xb, w1b.astype(bf), preferred_element_type=f32).astype(bf)
    childH = jnp.concatenate([w1t, XT], axis=0)
    G = jnp.concatenate([w1b.astype(bf), XB], axis=0)

    pc = pidx_ref[0]
    P = (pc == lax.broadcasted_iota(jnp.int32, (NPAD, NPAD), 1)).astype(bf)
    parentH = jnp.dot(P, G, preferred_element_type=f32)

    H = childH + parentH + b1_ref[...]
    Hact = jnp.where(H > 0, H, jnp.exp(jnp.minimum(H, 0.0)) - 1.0)
    out2 = jnp.dot(Hact, w2_ref[...], preferred_element_type=f32) + b2_ref[...]
    out2t = jnp.transpose(out2)
    mean = out2t[0:1, :]
    colmask = (lax.broadcasted_iota(jnp.int32, (1, NPAD), 1)
               < (NNODES - 1)).astype(f32)
    std = out2t[1:2, :] * colmask
    eps = eps_ref[0]
    samp_ref[0] = eps * jnp.exp(std) + mean - 2.0
    logq0 = jnp.sum((-0.5 * math.log(2 * math.pi) - 0.5 * eps * eps) * colmask)
    logq_ref[0] = jnp.full((1, 128), logq0 - jnp.sum(std), f32)


@jax.jit
def kernel(edge_index, W1m, b1m, W2m, b2m, W1s, b1s, W2s, b2s):
    f32 = jnp.float32
    bs = edge_index.shape[0]
    idx_fix = edge_index[:, NTIPS:, :]
    idx_fix = jnp.pad(idx_fix, ((0, 0), (0, NTIPS - DIM), (0, 0)))
    p_idx = edge_index[:, :NNODES - 1, 0]
    p_idx = jnp.pad(p_idx, ((0, 0), (0, NPAD - (NNODES - 1))))
    p_idx = p_idx[:, :, None]

    W1 = jnp.concatenate([W1m, W1s], axis=1)
    W1_top = W1[:NTIPS]
    W1_bot = W1[NTIPS:]
    b1 = jnp.concatenate([b1m, b1s])[None, :]
    W2 = jnp.zeros((2 * HID, 2), f32)
    W2 = W2.at[:HID, 0].set(W2m[:, 0]).at[HID:, 1].set(W2s[:, 0])
    b2 = jnp.stack([b2m[0], b2s[0]])[None, :]

    eps = jax.random.normal(jax.random.key(42), (bs, NNODES - 1), dtype=f32)
    eps_p = jnp.pad(eps, ((0, 0), (0, NPAD - (NNODES - 1))))[:, None, :]

    grid = (bs,)
    samp_out, logq_out = pl.pallas_call(
        _tree_kernel,
        grid=grid,
        in_specs=[
            pl.BlockSpec((1, NTIPS, 3), lambda b: (b, 0, 0)),
            pl.BlockSpec((1, NPAD, 1), lambda b: (b, 0, 0)),
            pl.BlockSpec((1, 1, NPAD), lambda b: (b, 0, 0)),
            pl.BlockSpec((NTIPS, NTIPS), lambda b: (0, 0)),
            pl.BlockSpec((NTIPS, NTIPS), lambda b: (0, 0)),
            pl.BlockSpec((1, 2 * HID), lambda b: (0, 0)),
            pl.BlockSpec((2 * HID, 2), lambda b: (0, 0)),
            pl.BlockSpec((1, 2), lambda b: (0, 0)),
        ],
        out_specs=[
            pl.BlockSpec((1, 1, NPAD), lambda b: (b, 0, 0)),
            pl.BlockSpec((1, 1, 128), lambda b: (b, 0, 0)),
        ],
        out_shape=[
            jax.ShapeDtypeStruct((bs, 1, NPAD), f32),
            jax.ShapeDtypeStruct((bs, 1, 128), f32),
        ],
    )(idx_fix, p_idx, eps_p, W1_top, W1_bot, b1, W2, b2)

    samp_log_branch = samp_out[:, 0, :NNODES - 1]
    logq_branch = logq_out[:, 0, 0]
    return samp_log_branch, logq_branch


# speedup vs baseline: 16.6100x; 1.1268x over previous
import functools
import math

import jax
import jax.numpy as jnp
from jax import lax
from jax.experimental import pallas as pl

NTIPS = 512
HID = 256
BS = 16
NNODES = 2 * NTIPS - 2
DIM = NTIPS - 2
NPAD = 1024
TOL = 1e-5
MAX_ITERS = 10000


def _tree_kernel(idx_ref, pidx_ref, eps_ref, w1t_ref, w1b_ref, b1_ref,
                 w2_ref, b2_ref, samp_ref, logq_ref):
    f32 = jnp.float32
    idxs = idx_ref[0]
    cols = lax.broadcasted_iota(jnp.int32, (NTIPS, NPAD), 1)
    cnt = jnp.zeros((NTIPS, NPAD), f32)
    for k in range(3):
        cnt = cnt + (idxs[:, k:k + 1] == cols).astype(f32)
    M = cnt * (1.0 / 3.0)
    C = M[:, :NTIPS]
    B = M[:, NTIPS:]

    X0 = jnp.full((NTIPS, NTIPS), 1.0 / NTIPS, f32)

    B2 = jnp.dot(B, B, preferred_element_type=f32)
    C2 = C + jnp.dot(B, C, preferred_element_type=f32)

    def cond_fn(carry):
        i, _, ln = carry
        return (i < MAX_ITERS) & (ln > TOL)

    def body_fn(carry):
        i, X, _ = carry
        X2 = C2 + jnp.dot(B2, X, preferred_element_type=f32)
        ln = jnp.sum(jnp.abs(X2 - X)) * (1.0 / (DIM * NTIPS))
        return i + 2, X2, ln

    _, X, _ = lax.while_loop(
        cond_fn, body_fn, (jnp.int32(0), X0, jnp.float32(jnp.inf)))

    bf = jnp.bfloat16
    w1t = w1t_ref[...]
    w1b = w1b_ref[...]
    Xb = X.astype(bf)
    XT = jnp.dot(Xb, w1t.astype(bf), preferred_element_type=f32)
    XB = jnp.dot(Xb, w1b.astype(bf), preferred_element_type=f32).astype(bf)
    childH = jnp.concatenate([w1t, XT], axis=0)
    G = jnp.concatenate([w1b.astype(bf), XB], axis=0)

    pc = pidx_ref[0]
    P = (pc == lax.broadcasted_iota(jnp.int32, (NPAD, NPAD), 1)).astype(bf)
    parentH = jnp.dot(P, G, preferred_element_type=f32)

    H = childH + parentH + b1_ref[...]
    Hact = jnp.where(H > 0, H, jnp.exp(jnp.minimum(H, 0.0)) - 1.0)
    out2 = jnp.dot(Hact, w2_ref[...], preferred_element_type=f32) + b2_ref[...]
    out2t = jnp.transpose(out2)
    mean = out2t[0:1, :]
    colmask = (lax.broadcasted_iota(jnp.int32, (1, NPAD), 1)
               < (NNODES - 1)).astype(f32)
    std = out2t[1:2, :] * colmask
    eps = eps_ref[0]
    samp_ref[0] = eps * jnp.exp(std) + mean - 2.0
    logq0 = jnp.sum((-0.5 * math.log(2 * math.pi) - 0.5 * eps * eps) * colmask)
    logq_ref[0] = jnp.full((1, 128), logq0 - jnp.sum(std), f32)


@jax.jit
def kernel(edge_index, W1m, b1m, W2m, b2m, W1s, b1s, W2s, b2s):
    f32 = jnp.float32
    bs = edge_index.shape[0]
    idx_fix = edge_index[:, NTIPS:, :]
    idx_fix = jnp.pad(idx_fix, ((0, 0), (0, NTIPS - DIM), (0, 0)))
    p_idx = edge_index[:, :NNODES - 1, 0]
    p_idx = jnp.pad(p_idx, ((0, 0), (0, NPAD - (NNODES - 1))))
    p_idx = p_idx[:, :, None]

    W1 = jnp.concatenate([W1m, W1s], axis=1)
    W1_top = W1[:NTIPS]
    W1_bot = W1[NTIPS:]
    b1 = jnp.concatenate([b1m, b1s])[None, :]
    W2 = jnp.zeros((2 * HID, 2), f32)
    W2 = W2.at[:HID, 0].set(W2m[:, 0]).at[HID:, 1].set(W2s[:, 0])
    b2 = jnp.stack([b2m[0], b2s[0]])[None, :]

    eps = jax.random.normal(jax.random.key(42), (bs, NNODES - 1), dtype=f32)
    eps_p = jnp.pad(eps, ((0, 0), (0, NPAD - (NNODES - 1))))[:, None, :]

    grid = (bs,)
    samp_out, logq_out = pl.pallas_call(
        _tree_kernel,
        grid=grid,
        in_specs=[
            pl.BlockSpec((1, NTIPS, 3), lambda b: (b, 0, 0)),
            pl.BlockSpec((1, NPAD, 1), lambda b: (b, 0, 0)),
            pl.BlockSpec((1, 1, NPAD), lambda b: (b, 0, 0)),
            pl.BlockSpec((NTIPS, NTIPS), lambda b: (0, 0)),
            pl.BlockSpec((NTIPS, NTIPS), lambda b: (0, 0)),
            pl.BlockSpec((1, 2 * HID), lambda b: (0, 0)),
            pl.BlockSpec((2 * HID, 2), lambda b: (0, 0)),
            pl.BlockSpec((1, 2), lambda b: (0, 0)),
        ],
        out_specs=[
            pl.BlockSpec((1, 1, NPAD), lambda b: (b, 0, 0)),
            pl.BlockSpec((1, 1, 128), lambda b: (b, 0, 0)),
        ],
        out_shape=[
            jax.ShapeDtypeStruct((bs, 1, NPAD), f32),
            jax.ShapeDtypeStruct((bs, 1, 128), f32),
        ],
    )(idx_fix, p_idx, eps_p, W1_top, W1_bot, b1, W2, b2)

    samp_log_branch = samp_out[:, 0, :NNODES - 1]
    logq_branch = logq_out[:, 0, 0]
    return samp_log_branch, logq_branch


# speedup vs baseline: 16.6732x; 1.0038x over previous
import functools
import math

import jax
import jax.numpy as jnp
from jax import lax
from jax.experimental import pallas as pl

NTIPS = 512
HID = 256
BS = 16
NNODES = 2 * NTIPS - 2
DIM = NTIPS - 2
NPAD = 1024
TOL = 1e-5
MAX_ITERS = 10000


def _tree_kernel(idx_ref, pidx_ref, eps_ref, w1t_ref, w1b_ref, b1_ref,
                 w2_ref, b2_ref, samp_ref, logq_ref):
    f32 = jnp.float32
    bf = jnp.bfloat16
    idxs = idx_ref[0]
    cols = lax.broadcasted_iota(jnp.int32, (NTIPS, NPAD), 1)
    cnt = jnp.zeros((NTIPS, NPAD), f32)
    for k in range(3):
        cnt = cnt + (idxs[:, k:k + 1] == cols).astype(f32)
    Kc = cnt[:, :NTIPS]
    K = cnt[:, NTIPS:]

    X0 = jnp.full((NTIPS, NTIPS), 1.0 / NTIPS, f32)

    Kb = K.astype(bf)
    B2 = jnp.dot(Kb, Kb, preferred_element_type=f32) * (1.0 / 9.0)
    C2 = (Kc * (1.0 / 3.0)
          + jnp.dot(Kb, Kc.astype(bf), preferred_element_type=f32) * (1.0 / 9.0))

    def cond_fn(carry):
        i, _, ln = carry
        return (i < MAX_ITERS) & (ln > TOL)

    def body_fn(carry):
        i, X, _ = carry
        X2 = C2 + jnp.dot(B2, X, preferred_element_type=f32)
        ln = jnp.sum(jnp.abs(X2 - X)) * (1.0 / (DIM * NTIPS))
        return i + 2, X2, ln

    _, X, _ = lax.while_loop(
        cond_fn, body_fn, (jnp.int32(0), X0, jnp.float32(jnp.inf)))

    w1t = w1t_ref[...]
    w1b = w1b_ref[...]
    Xb = X.astype(bf)
    XT = jnp.dot(Xb, w1t.astype(bf), preferred_element_type=f32)
    XB = jnp.dot(Xb, w1b.astype(bf), preferred_element_type=f32).astype(bf)
    childH = jnp.concatenate([w1t, XT], axis=0)
    G = jnp.concatenate([w1b.astype(bf), XB], axis=0)

    pc = pidx_ref[0]
    P = (pc == lax.broadcasted_iota(jnp.int32, (NPAD, NPAD), 1)).astype(bf)
    parentH = jnp.dot(P, G, preferred_element_type=f32)

    H = childH + parentH + b1_ref[...]
    Hact = jnp.where(H > 0, H, jnp.exp(jnp.minimum(H, 0.0)) - 1.0)
    out2 = jnp.dot(Hact, w2_ref[...], preferred_element_type=f32) + b2_ref[...]
    out2t = jnp.transpose(out2)
    mean = out2t[0:1, :]
    colmask = (lax.broadcasted_iota(jnp.int32, (1, NPAD), 1)
               < (NNODES - 1)).astype(f32)
    std = out2t[1:2, :] * colmask
    eps = eps_ref[0]
    samp_ref[0] = eps * jnp.exp(std) + mean - 2.0
    logq0 = jnp.sum((-0.5 * math.log(2 * math.pi) - 0.5 * eps * eps) * colmask)
    logq_ref[0] = jnp.full((1, 128), logq0 - jnp.sum(std), f32)


@jax.jit
def kernel(edge_index, W1m, b1m, W2m, b2m, W1s, b1s, W2s, b2s):
    f32 = jnp.float32
    bs = edge_index.shape[0]
    idx_fix = edge_index[:, NTIPS:, :]
    idx_fix = jnp.pad(idx_fix, ((0, 0), (0, NTIPS - DIM), (0, 0)))
    p_idx = edge_index[:, :NNODES - 1, 0]
    p_idx = jnp.pad(p_idx, ((0, 0), (0, NPAD - (NNODES - 1))))
    p_idx = p_idx[:, :, None]

    W1 = jnp.concatenate([W1m, W1s], axis=1)
    W1_top = W1[:NTIPS]
    W1_bot = W1[NTIPS:]
    b1 = jnp.concatenate([b1m, b1s])[None, :]
    W2 = jnp.zeros((2 * HID, 2), f32)
    W2 = W2.at[:HID, 0].set(W2m[:, 0]).at[HID:, 1].set(W2s[:, 0])
    b2 = jnp.stack([b2m[0], b2s[0]])[None, :]

    eps = jax.random.normal(jax.random.key(42), (bs, NNODES - 1), dtype=f32)
    eps_p = jnp.pad(eps, ((0, 0), (0, NPAD - (NNODES - 1))))[:, None, :]

    grid = (bs,)
    samp_out, logq_out = pl.pallas_call(
        _tree_kernel,
        grid=grid,
        in_specs=[
            pl.BlockSpec((1, NTIPS, 3), lambda b: (b, 0, 0)),
            pl.BlockSpec((1, NPAD, 1), lambda b: (b, 0, 0)),
            pl.BlockSpec((1, 1, NPAD), lambda b: (b, 0, 0)),
            pl.BlockSpec((NTIPS, NTIPS), lambda b: (0, 0)),
            pl.BlockSpec((NTIPS, NTIPS), lambda b: (0, 0)),
            pl.BlockSpec((1, 2 * HID), lambda b: (0, 0)),
            pl.BlockSpec((2 * HID, 2), lambda b: (0, 0)),
            pl.BlockSpec((1, 2), lambda b: (0, 0)),
        ],
        out_specs=[
            pl.BlockSpec((1, 1, NPAD), lambda b: (b, 0, 0)),
            pl.BlockSpec((1, 1, 128), lambda b: (b, 0, 0)),
        ],
        out_shape=[
            jax.ShapeDtypeStruct((bs, 1, NPAD), f32),
            jax.ShapeDtypeStruct((bs, 1, 128), f32),
        ],
    )(idx_fix, p_idx, eps_p, W1_top, W1_bot, b1, W2, b2)

    samp_log_branch = samp_out[:, 0, :NNODES - 1]
    logq_branch = logq_out[:, 0, 0]
    return samp_log_branch, logq_branch


# speedup vs baseline: 18.0038x; 1.0798x over previous
import functools
import math

import jax
import jax.numpy as jnp
from jax import lax
from jax.experimental import pallas as pl

NTIPS = 512
HID = 256
BS = 16
NNODES = 2 * NTIPS - 2
DIM = NTIPS - 2
NPAD = 1024
TOL = 1e-5
MAX_ITERS = 10000


def _tree_kernel(idx_ref, pidx_ref, eps_ref, w1t_ref, w1b_ref, b1_ref,
                 w2_ref, b2_ref, samp_ref, logq_ref):
    f32 = jnp.float32
    bf = jnp.bfloat16
    idxs = idx_ref[0]
    cols = lax.broadcasted_iota(jnp.int32, (NTIPS, NPAD), 1)
    cnt = jnp.zeros((NTIPS, NPAD), f32)
    for k in range(3):
        cnt = cnt + (idxs[:, k:k + 1] == cols).astype(f32)
    Kc = cnt[:, :NTIPS]
    K = cnt[:, NTIPS:]

    X0 = jnp.full((NTIPS, NTIPS), 1.0 / NTIPS, f32)

    Kb = K.astype(bf)
    B2 = jnp.dot(Kb, Kb, preferred_element_type=f32) * (1.0 / 9.0)
    C2 = (Kc * (1.0 / 3.0)
          + jnp.dot(Kb, Kc.astype(bf), preferred_element_type=f32) * (1.0 / 9.0))

    def cond_fn(carry):
        i, _, ln = carry
        return (i < MAX_ITERS) & (ln > TOL)

    def body_fn(carry):
        i, X, _ = carry
        X1 = C2 + jnp.dot(B2, X, preferred_element_type=f32)
        X2 = C2 + jnp.dot(B2, X1, preferred_element_type=f32)
        ln = jnp.sum(jnp.abs(X2 - X1)) * (1.0 / (DIM * NTIPS))
        return i + 4, X2, ln

    _, X, _ = lax.while_loop(
        cond_fn, body_fn, (jnp.int32(0), X0, jnp.float32(jnp.inf)))

    w1t = w1t_ref[...]
    w1b = w1b_ref[...]
    Xb = X.astype(bf)
    XT = jnp.dot(Xb, w1t.astype(bf), preferred_element_type=f32)
    XB = jnp.dot(Xb, w1b.astype(bf), preferred_element_type=f32).astype(bf)
    childH = jnp.concatenate([w1t, XT], axis=0)
    G = jnp.concatenate([w1b.astype(bf), XB], axis=0)

    pc = pidx_ref[0]
    P = (pc == lax.broadcasted_iota(jnp.int32, (NPAD, NPAD), 1)).astype(bf)
    parentH = jnp.dot(P, G, preferred_element_type=f32)

    H = childH + parentH + b1_ref[...]
    Hact = jnp.where(H > 0, H, jnp.exp(jnp.minimum(H, 0.0)) - 1.0)
    out2 = jnp.dot(Hact, w2_ref[...], preferred_element_type=f32) + b2_ref[...]
    out2t = jnp.transpose(out2)
    mean = out2t[0:1, :]
    colmask = (lax.broadcasted_iota(jnp.int32, (1, NPAD), 1)
               < (NNODES - 1)).astype(f32)
    std = out2t[1:2, :] * colmask
    eps = eps_ref[0]
    samp_ref[0] = eps * jnp.exp(std) + mean - 2.0
    logq0 = jnp.sum((-0.5 * math.log(2 * math.pi) - 0.5 * eps * eps) * colmask)
    logq_ref[0] = jnp.full((1, 128), logq0 - jnp.sum(std), f32)


@jax.jit
def kernel(edge_index, W1m, b1m, W2m, b2m, W1s, b1s, W2s, b2s):
    f32 = jnp.float32
    bs = edge_index.shape[0]
    idx_fix = edge_index[:, NTIPS:, :]
    idx_fix = jnp.pad(idx_fix, ((0, 0), (0, NTIPS - DIM), (0, 0)))
    p_idx = edge_index[:, :NNODES - 1, 0]
    p_idx = jnp.pad(p_idx, ((0, 0), (0, NPAD - (NNODES - 1))))
    p_idx = p_idx[:, :, None]

    W1 = jnp.concatenate([W1m, W1s], axis=1)
    W1_top = W1[:NTIPS]
    W1_bot = W1[NTIPS:]
    b1 = jnp.concatenate([b1m, b1s])[None, :]
    W2 = jnp.zeros((2 * HID, 2), f32)
    W2 = W2.at[:HID, 0].set(W2m[:, 0]).at[HID:, 1].set(W2s[:, 0])
    b2 = jnp.stack([b2m[0], b2s[0]])[None, :]

    eps = jax.random.normal(jax.random.key(42), (bs, NNODES - 1), dtype=f32)
    eps_p = jnp.pad(eps, ((0, 0), (0, NPAD - (NNODES - 1))))[:, None, :]

    grid = (bs,)
    samp_out, logq_out = pl.pallas_call(
        _tree_kernel,
        grid=grid,
        in_specs=[
            pl.BlockSpec((1, NTIPS, 3), lambda b: (b, 0, 0)),
            pl.BlockSpec((1, NPAD, 1), lambda b: (b, 0, 0)),
            pl.BlockSpec((1, 1, NPAD), lambda b: (b, 0, 0)),
            pl.BlockSpec((NTIPS, NTIPS), lambda b: (0, 0)),
            pl.BlockSpec((NTIPS, NTIPS), lambda b: (0, 0)),
            pl.BlockSpec((1, 2 * HID), lambda b: (0, 0)),
            pl.BlockSpec((2 * HID, 2), lambda b: (0, 0)),
            pl.BlockSpec((1, 2), lambda b: (0, 0)),
        ],
        out_specs=[
            pl.BlockSpec((1, 1, NPAD), lambda b: (b, 0, 0)),
            pl.BlockSpec((1, 1, 128), lambda b: (b, 0, 0)),
        ],
        out_shape=[
            jax.ShapeDtypeStruct((bs, 1, NPAD), f32),
            jax.ShapeDtypeStruct((bs, 1, 128), f32),
        ],
    )(idx_fix, p_idx, eps_p, W1_top, W1_bot, b1, W2, b2)

    samp_log_branch = samp_out[:, 0, :NNODES - 1]
    logq_branch = logq_out[:, 0, 0]
    return samp_log_branch, logq_branch
